# baseline (speedup 1.0000x reference)
import functools
import jax
import jax.numpy as jnp
from jax import lax
from jax.experimental import pallas as pl
from jax.experimental.pallas import tpu as pltpu
from jax.experimental.pallas import tpu_sc as plsc

_N = 4194304
_NC = 2
_NS = 16
_NW = _NC * _NS
_C = _N // _NW
_BLK = 16384
_NBLK = _C // _BLK

_mesh = plsc.VectorSubcoreMesh(core_axis_name="c", subcore_axis_name="s")


@functools.partial(
    pl.kernel,
    mesh=_mesh,
    compiler_params=pltpu.CompilerParams(needs_layout_passes=False),
    out_type=jax.ShapeDtypeStruct((_N,), jnp.float32),
    scratch_types=[
        pltpu.VMEM((_BLK,), jnp.int32),
        pltpu.VMEM((_BLK,), jnp.float32),
        pltpu.VMEM((16,), jnp.float32),
        pltpu.VMEM((16,), jnp.float32),
    ],
)
def _lut_kernel(x_hbm, w_hbm, b_hbm, out_hbm, x_v, y_v, tbl_v, b_v):
    pltpu.sync_copy(w_hbm, tbl_v)
    pltpu.sync_copy(b_hbm, b_v)
    tbl_v[...] = jnp.clip(tbl_v[...] + b_v[...], 0.01, 1.0)

    wid = lax.axis_index("s") * _NC + lax.axis_index("c")
    base = wid * _C

    def blk_body(i, carry):
        off = base + i * _BLK
        pltpu.sync_copy(x_hbm.at[pl.ds(off, _BLK)], x_v)

        def inner(j, c):
            j16 = pl.multiple_of(j * 16, 16)
            idx = x_v[pl.ds(j16, 16)]
            y_v[pl.ds(j16, 16)] = plsc.load_gather(tbl_v, [idx])
            return c

        lax.fori_loop(0, _BLK // 16, inner, 0)
        pltpu.sync_copy(y_v, out_hbm.at[pl.ds(off, _BLK)])
        return carry

    lax.fori_loop(0, _NBLK, blk_body, 0)


def kernel(x, W, b):
    w16 = jnp.pad(W.reshape(15), (0, 1))
    b16 = jnp.broadcast_to(b, (16,))
    y = _lut_kernel(x, w16, b16)
    return y.reshape(_N, 1)


# speedup vs baseline: 2.4167x; 2.4167x over previous
import functools
import jax
import jax.numpy as jnp
from jax import lax
from jax.experimental import pallas as pl
from jax.experimental.pallas import tpu as pltpu
from jax.experimental.pallas import tpu_sc as plsc

_N = 4194304
_NC = 2
_NS = 16
_NW = _NC * _NS
_C = _N // _NW
_BLK = 16384
_NBLK = _C // _BLK

_mesh = plsc.VectorSubcoreMesh(core_axis_name="c", subcore_axis_name="s")


@functools.partial(
    pl.kernel,
    mesh=_mesh,
    compiler_params=pltpu.CompilerParams(needs_layout_passes=False),
    out_type=jax.ShapeDtypeStruct((_N,), jnp.float32),
    scratch_types=[
        pltpu.VMEM((_BLK,), jnp.int32),
        pltpu.VMEM((_BLK,), jnp.int32),
        pltpu.VMEM((_BLK,), jnp.float32),
        pltpu.VMEM((_BLK,), jnp.float32),
        pltpu.VMEM((16,), jnp.float32),
        pltpu.VMEM((16,), jnp.float32),
        pltpu.SemaphoreType.DMA,
        pltpu.SemaphoreType.DMA,
        pltpu.SemaphoreType.DMA,
        pltpu.SemaphoreType.DMA,
    ],
)
def _lut_kernel(x_hbm, w_hbm, b_hbm, out_hbm,
                x0_v, x1_v, y0_v, y1_v, tbl_v, b_v,
                in_sem0, in_sem1, out_sem0, out_sem1):
    pltpu.sync_copy(w_hbm, tbl_v)
    pltpu.sync_copy(b_hbm, b_v)
    tbl_v[...] = jnp.clip(tbl_v[...] + b_v[...], 0.01, 1.0)

    wid = lax.axis_index("s") * _NC + lax.axis_index("c")
    base = wid * _C

    xb = [x0_v, x1_v]
    yb = [y0_v, y1_v]
    in_sems = [in_sem0, in_sem1]
    out_sems = [out_sem0, out_sem1]

    in_copies = [None, None]
    out_copies = [None, None]

    def start_in(i):
        s = i % 2
        off = base + i * _BLK
        in_copies[s] = pltpu.async_copy(
            x_hbm.at[pl.ds(off, _BLK)], xb[s], in_sems[s])

    start_in(0)
    for i in range(_NBLK):
        s = i % 2
        if i + 1 < _NBLK:
            start_in(i + 1)
        in_copies[s].wait()
        if out_copies[s] is not None:
            out_copies[s].wait()

        x_ref = xb[s]
        y_ref = yb[s]

        @plsc.parallel_loop(0, _BLK, step=16, unroll=8)
        def _(j):
            j16 = pl.multiple_of(j, 16)
            y_ref[pl.ds(j16, 16)] = plsc.load_gather(
                tbl_v, [x_ref[pl.ds(j16, 16)]])

        off = base + i * _BLK
        out_copies[s] = pltpu.async_copy(
            y_ref, out_hbm.at[pl.ds(off, _BLK)], out_sems[s])

    out_copies[0].wait()
    out_copies[1].wait()


def kernel(x, W, b):
    w16 = jnp.pad(W.reshape(15), (0, 1))
    b16 = jnp.broadcast_to(b, (16,))
    y = _lut_kernel(x, w16, b16)
    return y.reshape(_N, 1)
